# Initial kernel scaffold; baseline (speedup 1.0000x reference)
import jax
import jax.numpy as jnp
from jax.experimental import pallas as pl


def kernel(z, W):
    raise NotImplementedError("write your pallas kernel here")


# speedup vs baseline: 88.5916x; 88.5916x over previous
import functools

import jax
import jax.numpy as jnp
from jax.experimental import pallas as pl
from jax.experimental.pallas import tpu as pltpu
from jax.experimental.pallas import tpu_sc as plsc

N_TOK = 4608
D = 256
K = 8192
BETA = 0.01
KT = 512
KE = 1024
NEG_INF = -3.0e38


def _rownorm(x):
    n = jnp.sqrt(jnp.sum(x * x, axis=1, keepdims=True))
    return x / jnp.maximum(n, 1e-12)


def _argmax_body(zt_ref, w_ref, zn_ref, idx_ref, znn_ref, gmax_ref, gidx_ref):
    k = pl.program_id(0)

    @pl.when(k == 0)
    def _init():
        x = zt_ref[...]
        zn = _rownorm(x)
        zn_ref[...] = zn
        znn_ref[...] = _rownorm(zn).astype(jnp.bfloat16)
        gmax_ref[...] = jnp.full((N_TOK, 1), NEG_INF, jnp.float32)
        gidx_ref[...] = jnp.zeros((N_TOK, 1), jnp.int32)

    wt = w_ref[...]
    wnn = _rownorm(_rownorm(wt))
    d = jax.lax.dot_general(
        znn_ref[...], wnn.astype(jnp.bfloat16),
        (((1,), (1,)), ((), ())),
        preferred_element_type=jnp.float32,
    )
    lmax = jnp.max(d, axis=1, keepdims=True)
    col = jax.lax.broadcasted_iota(jnp.int32, d.shape, 1)
    lidx = jnp.max(jnp.where(d == lmax, col, -1), axis=1, keepdims=True)
    upd = lmax >= gmax_ref[...]
    gmax_ref[...] = jnp.where(upd, lmax, gmax_ref[...])
    gidx_ref[...] = jnp.where(upd, lidx + k * KT, gidx_ref[...])

    @pl.when(k == pl.num_programs(0) - 1)
    def _fin():
        idx_ref[...] = gidx_ref[...]


def _run_argmax(zt, w, interpret=False):
    return pl.pallas_call(
        _argmax_body,
        grid=(K // KT,),
        in_specs=[
            pl.BlockSpec((N_TOK, D), lambda k: (0, 0)),
            pl.BlockSpec((KT, D), lambda k: (k, 0)),
        ],
        out_specs=[
            pl.BlockSpec((N_TOK, D), lambda k: (0, 0)),
            pl.BlockSpec((N_TOK, 1), lambda k: (0, 0)),
        ],
        out_shape=[
            jax.ShapeDtypeStruct((N_TOK, D), jnp.float32),
            jax.ShapeDtypeStruct((N_TOK, 1), jnp.int32),
        ],
        scratch_shapes=[
            pltpu.VMEM((N_TOK, D), jnp.bfloat16),
            pltpu.VMEM((N_TOK, 1), jnp.float32),
            pltpu.VMEM((N_TOK, 1), jnp.int32),
        ],
        interpret=interpret,
    )(zt, w)


GATHER_WIN = 128


def _sc_gather(w, idx_flat):
    mesh = plsc.VectorSubcoreMesh(core_axis_name="core", subcore_axis_name="subcore")

    @pl.kernel(out_type=jax.ShapeDtypeStruct((N_TOK, D), jnp.float32), mesh=mesh)
    def gather_kernel(w_hbm, i_hbm, o_hbm):
        def body(i_vmem, o_vmem):
            pltpu.sync_copy(w_hbm.at[i_vmem.at[0]], o_vmem)

        pltpu.emit_pipeline(
            body,
            grid=(N_TOK // GATHER_WIN,),
            in_specs=[pl.BlockSpec((1, GATHER_WIN), index_map=lambda i: (0, i))],
            out_specs=[pl.BlockSpec((GATHER_WIN, D), index_map=lambda i: (i, 0))],
            core_axis_name="subcore",
            dimension_semantics=(pltpu.PARALLEL,),
        )(i_hbm, o_hbm)

    return gather_kernel(w, idx_flat)


def _onehot_body(idx_ref, enc_ref, cnt_ref):
    j = pl.program_id(0)
    col = jax.lax.broadcasted_iota(jnp.int32, (N_TOK, KE), 1) + j * KE
    oh = jnp.where(idx_ref[...] == col, 1.0, 0.0).astype(jnp.float32)
    enc_ref[...] = oh
    cnt_ref[...] = jnp.sum(oh, axis=0, keepdims=True)


def _run_onehot(idx, interpret=False):
    return pl.pallas_call(
        _onehot_body,
        grid=(K // KE,),
        in_specs=[pl.BlockSpec((N_TOK, 1), lambda j: (0, 0))],
        out_specs=[
            pl.BlockSpec((N_TOK, KE), lambda j: (0, j)),
            pl.BlockSpec((1, KE), lambda j: (0, j)),
        ],
        out_shape=[
            jax.ShapeDtypeStruct((N_TOK, K), jnp.float32),
            jax.ShapeDtypeStruct((1, K), jnp.float32),
        ],
        compiler_params=pltpu.CompilerParams(
            dimension_semantics=("parallel",),
        ),
        interpret=interpret,
    )(idx)


def _final_body(zq_raw_ref, zn_ref, cnt_ref, zq_ref, loss_ref, ppx_ref):
    zn = zn_ref[...]
    zqn = _rownorm(zq_raw_ref[...])
    zq_ref[...] = zn + (zqn - zn)
    diff = zqn - zn
    m = jnp.sum(diff * diff) / (N_TOK * D)
    loss_ref[0, 0] = BETA * m + m
    p = cnt_ref[...] / N_TOK
    ppx_ref[0, 0] = jnp.exp(-jnp.sum(p * jnp.log(p + 1e-10)))


def _run_final(zq_raw, zn, cnt, interpret=False):
    return pl.pallas_call(
        _final_body,
        in_specs=[
            pl.BlockSpec((N_TOK, D), lambda: (0, 0)),
            pl.BlockSpec((N_TOK, D), lambda: (0, 0)),
            pl.BlockSpec((1, K), lambda: (0, 0)),
        ],
        out_specs=[
            pl.BlockSpec((N_TOK, D), lambda: (0, 0)),
            pl.BlockSpec((1, 1), memory_space=pltpu.SMEM),
            pl.BlockSpec((1, 1), memory_space=pltpu.SMEM),
        ],
        out_shape=[
            jax.ShapeDtypeStruct((N_TOK, D), jnp.float32),
            jax.ShapeDtypeStruct((1, 1), jnp.float32),
            jax.ShapeDtypeStruct((1, 1), jnp.float32),
        ],
        interpret=interpret,
    )(zq_raw, zn, cnt)


def kernel(z, W):
    zt = jnp.transpose(z, (0, 2, 3, 1)).reshape(N_TOK, D)
    zn, idx = _run_argmax(zt, W)
    zq_raw = _sc_gather(W, idx.reshape(1, N_TOK))
    encodings, counts = _run_onehot(idx)
    zq, loss, ppx = _run_final(zq_raw, zn, counts)
    z_q = jnp.transpose(zq.reshape(8, 24, 24, D), (0, 3, 1, 2))
    return (z_q, idx.reshape(N_TOK), loss.reshape(()), encodings, ppx.reshape(()))


# speedup vs baseline: 103.6616x; 1.1701x over previous
import functools

import jax
import jax.numpy as jnp
from jax.experimental import pallas as pl
from jax.experimental.pallas import tpu as pltpu
from jax.experimental.pallas import tpu_sc as plsc

N_TOK = 4608
D = 256
K = 8192
BETA = 0.01
KT = 512
KE = 1024
NEG_INF = -3.0e38


def _rownorm(x):
    n = jnp.sqrt(jnp.sum(x * x, axis=1, keepdims=True))
    return x / jnp.maximum(n, 1e-12)


def _argmax_body(zt_ref, w_ref, zn_ref, idx_ref, znn_ref, vmax_ref, vidx_ref):
    k = pl.program_id(0)

    @pl.when(k == 0)
    def _init():
        x = zt_ref[...]
        zn = _rownorm(x)
        zn_ref[...] = zn
        znn_ref[...] = _rownorm(zn).astype(jnp.bfloat16)
        vmax_ref[...] = jnp.full((N_TOK, 128), NEG_INF, jnp.float32)
        vidx_ref[...] = jnp.zeros((N_TOK, 128), jnp.int32)

    wt = w_ref[...]
    wnn = _rownorm(_rownorm(wt))
    d = jax.lax.dot_general(
        znn_ref[...], wnn.astype(jnp.bfloat16),
        (((1,), (1,)), ((), ())),
        preferred_element_type=jnp.float32,
    )
    lane = jax.lax.broadcasted_iota(jnp.int32, (N_TOK, 128), 1)
    for g in range(KT // 128):
        dg = d[:, g * 128:(g + 1) * 128]
        upd = dg >= vmax_ref[...]
        vidx_ref[...] = jnp.where(upd, lane + (k * KT + g * 128), vidx_ref[...])
        vmax_ref[...] = jnp.maximum(dg, vmax_ref[...])

    @pl.when(k == pl.num_programs(0) - 1)
    def _fin():
        vm = vmax_ref[...]
        m1 = jnp.max(vm, axis=1, keepdims=True)
        cand = jnp.where(vm == m1, vidx_ref[...], -1)
        idx_ref[...] = jnp.max(cand, axis=1, keepdims=True)


def _run_argmax(zt, w, interpret=False):
    return pl.pallas_call(
        _argmax_body,
        grid=(K // KT,),
        in_specs=[
            pl.BlockSpec((N_TOK, D), lambda k: (0, 0)),
            pl.BlockSpec((KT, D), lambda k: (k, 0)),
        ],
        out_specs=[
            pl.BlockSpec((N_TOK, D), lambda k: (0, 0)),
            pl.BlockSpec((N_TOK, 1), lambda k: (0, 0)),
        ],
        out_shape=[
            jax.ShapeDtypeStruct((N_TOK, D), jnp.float32),
            jax.ShapeDtypeStruct((N_TOK, 1), jnp.int32),
        ],
        scratch_shapes=[
            pltpu.VMEM((N_TOK, D), jnp.bfloat16),
            pltpu.VMEM((N_TOK, 128), jnp.float32),
            pltpu.VMEM((N_TOK, 128), jnp.int32),
        ],
        interpret=interpret,
    )(zt, w)


GATHER_WIN = 128


def _sc_gather(w, idx_flat):
    mesh = plsc.VectorSubcoreMesh(core_axis_name="core", subcore_axis_name="subcore")

    @pl.kernel(out_type=jax.ShapeDtypeStruct((N_TOK, D), jnp.float32), mesh=mesh)
    def gather_kernel(w_hbm, i_hbm, o_hbm):
        def body(i_vmem, o_vmem):
            pltpu.sync_copy(w_hbm.at[i_vmem.at[0]], o_vmem)

        pltpu.emit_pipeline(
            body,
            grid=(N_TOK // GATHER_WIN,),
            in_specs=[pl.BlockSpec((1, GATHER_WIN), index_map=lambda i: (0, i))],
            out_specs=[pl.BlockSpec((GATHER_WIN, D), index_map=lambda i: (i, 0))],
            core_axis_name="subcore",
            dimension_semantics=(pltpu.PARALLEL,),
        )(i_hbm, o_hbm)

    return gather_kernel(w, idx_flat)


def _onehot_body(idx_ref, enc_ref, cnt_ref):
    j = pl.program_id(0)
    col = jax.lax.broadcasted_iota(jnp.int32, (N_TOK, KE), 1) + j * KE
    oh = jnp.where(idx_ref[...] == col, 1.0, 0.0).astype(jnp.float32)
    enc_ref[...] = oh
    cnt_ref[...] = jnp.sum(oh, axis=0, keepdims=True)


def _run_onehot(idx, interpret=False):
    return pl.pallas_call(
        _onehot_body,
        grid=(K // KE,),
        in_specs=[pl.BlockSpec((N_TOK, 1), lambda j: (0, 0))],
        out_specs=[
            pl.BlockSpec((N_TOK, KE), lambda j: (0, j)),
            pl.BlockSpec((1, KE), lambda j: (0, j)),
        ],
        out_shape=[
            jax.ShapeDtypeStruct((N_TOK, K), jnp.float32),
            jax.ShapeDtypeStruct((1, K), jnp.float32),
        ],
        compiler_params=pltpu.CompilerParams(
            dimension_semantics=("parallel",),
        ),
        interpret=interpret,
    )(idx)


def _final_body(zq_raw_ref, zn_ref, cnt_ref, zq_ref, loss_ref, ppx_ref):
    zn = zn_ref[...]
    zqn = _rownorm(zq_raw_ref[...])
    zq_ref[...] = zn + (zqn - zn)
    diff = zqn - zn
    m = jnp.sum(diff * diff) / (N_TOK * D)
    loss_ref[0, 0] = BETA * m + m
    p = cnt_ref[...] / N_TOK
    ppx_ref[0, 0] = jnp.exp(-jnp.sum(p * jnp.log(p + 1e-10)))


def _run_final(zq_raw, zn, cnt, interpret=False):
    return pl.pallas_call(
        _final_body,
        in_specs=[
            pl.BlockSpec((N_TOK, D), lambda: (0, 0)),
            pl.BlockSpec((N_TOK, D), lambda: (0, 0)),
            pl.BlockSpec((1, K), lambda: (0, 0)),
        ],
        out_specs=[
            pl.BlockSpec((N_TOK, D), lambda: (0, 0)),
            pl.BlockSpec((1, 1), memory_space=pltpu.SMEM),
            pl.BlockSpec((1, 1), memory_space=pltpu.SMEM),
        ],
        out_shape=[
            jax.ShapeDtypeStruct((N_TOK, D), jnp.float32),
            jax.ShapeDtypeStruct((1, 1), jnp.float32),
            jax.ShapeDtypeStruct((1, 1), jnp.float32),
        ],
        interpret=interpret,
    )(zq_raw, zn, cnt)


def kernel(z, W):
    zt = jnp.transpose(z, (0, 2, 3, 1)).reshape(N_TOK, D)
    zn, idx = _run_argmax(zt, W)
    zq_raw = _sc_gather(W, idx.reshape(1, N_TOK))
    encodings, counts = _run_onehot(idx)
    zq, loss, ppx = _run_final(zq_raw, zn, counts)
    z_q = jnp.transpose(zq.reshape(8, 24, 24, D), (0, 3, 1, 2))
    return (z_q, idx.reshape(N_TOK), loss.reshape(()), encodings, ppx.reshape(()))


# speedup vs baseline: 107.1761x; 1.0339x over previous
import functools

import jax
import jax.numpy as jnp
from jax.experimental import pallas as pl
from jax.experimental.pallas import tpu as pltpu
from jax.experimental.pallas import tpu_sc as plsc

N_TOK = 4608
D = 256
K = 8192
BETA = 0.01
KT = 1024
NB = 576
NEG_INF = -3.0e38


def _rownorm(x):
    n = jnp.sqrt(jnp.sum(x * x, axis=1, keepdims=True))
    return x / jnp.maximum(n, 1e-12)


def _argmax_body(zt_ref, w_ref, zn_ref, idx_ref, znn_ref, vmax_ref, vidx_ref):
    k = pl.program_id(0)

    @pl.when(k == 0)
    def _init():
        x = zt_ref[...]
        zn = _rownorm(x)
        zn_ref[...] = zn
        znn_ref[...] = _rownorm(zn).astype(jnp.bfloat16)
        vmax_ref[...] = jnp.full((N_TOK, 128), NEG_INF, jnp.float32)
        vidx_ref[...] = jnp.zeros((N_TOK, 128), jnp.int32)

    wt = w_ref[...]
    wnn = _rownorm(_rownorm(wt))
    d = jax.lax.dot_general(
        znn_ref[...], wnn.astype(jnp.bfloat16),
        (((1,), (1,)), ((), ())),
        preferred_element_type=jnp.float32,
    )
    lane = jax.lax.broadcasted_iota(jnp.int32, (N_TOK, 128), 1)
    for g in range(KT // 128):
        dg = d[:, g * 128:(g + 1) * 128]
        upd = dg >= vmax_ref[...]
        vidx_ref[...] = jnp.where(upd, lane + (k * KT + g * 128), vidx_ref[...])
        vmax_ref[...] = jnp.maximum(dg, vmax_ref[...])

    @pl.when(k == pl.num_programs(0) - 1)
    def _fin():
        vm = vmax_ref[...]
        m1 = jnp.max(vm, axis=1, keepdims=True)
        cand = jnp.where(vm == m1, vidx_ref[...], -1)
        idx_ref[...] = jnp.max(cand, axis=1, keepdims=True)


def _run_argmax(zt, w, interpret=False):
    return pl.pallas_call(
        _argmax_body,
        grid=(K // KT,),
        in_specs=[
            pl.BlockSpec((N_TOK, D), lambda k: (0, 0)),
            pl.BlockSpec((KT, D), lambda k: (k, 0)),
        ],
        out_specs=[
            pl.BlockSpec((N_TOK, D), lambda k: (0, 0)),
            pl.BlockSpec((N_TOK, 1), lambda k: (0, 0)),
        ],
        out_shape=[
            jax.ShapeDtypeStruct((N_TOK, D), jnp.float32),
            jax.ShapeDtypeStruct((N_TOK, 1), jnp.int32),
        ],
        scratch_shapes=[
            pltpu.VMEM((N_TOK, D), jnp.bfloat16),
            pltpu.VMEM((N_TOK, 128), jnp.float32),
            pltpu.VMEM((N_TOK, 128), jnp.int32),
        ],
        interpret=interpret,
    )(zt, w)


GATHER_WIN = 128


def _sc_gather(w, idx_flat):
    mesh = plsc.VectorSubcoreMesh(core_axis_name="core", subcore_axis_name="subcore")

    @pl.kernel(out_type=jax.ShapeDtypeStruct((N_TOK, D), jnp.float32), mesh=mesh)
    def gather_kernel(w_hbm, i_hbm, o_hbm):
        def body(i_vmem, o_vmem):
            pltpu.sync_copy(w_hbm.at[i_vmem.at[0]], o_vmem)

        pltpu.emit_pipeline(
            body,
            grid=(N_TOK // GATHER_WIN,),
            in_specs=[pl.BlockSpec((1, GATHER_WIN), index_map=lambda i: (0, i))],
            out_specs=[pl.BlockSpec((GATHER_WIN, D), index_map=lambda i: (i, 0))],
            core_axis_name="subcore",
            dimension_semantics=(pltpu.PARALLEL,),
        )(i_hbm, o_hbm)

    return gather_kernel(w, idx_flat)


def _onehot_body(idx_ref, enc_ref, cnt_ref, acc_ref):
    i = pl.program_id(0)
    col = jax.lax.broadcasted_iota(jnp.int32, (NB, K), 1)
    oh = jnp.where(idx_ref[...] == col, 1.0, 0.0).astype(jnp.float32)
    enc_ref[...] = oh
    part = jnp.sum(oh, axis=0, keepdims=True)

    @pl.when(i == 0)
    def _init():
        acc_ref[...] = part

    @pl.when(i > 0)
    def _acc():
        acc_ref[...] += part

    @pl.when(i == pl.num_programs(0) - 1)
    def _fin():
        cnt_ref[...] = acc_ref[...]


def _run_onehot(idx, interpret=False):
    return pl.pallas_call(
        _onehot_body,
        grid=(N_TOK // NB,),
        in_specs=[pl.BlockSpec((NB, 1), lambda i: (i, 0))],
        out_specs=[
            pl.BlockSpec((NB, K), lambda i: (i, 0)),
            pl.BlockSpec((1, K), lambda i: (0, 0)),
        ],
        out_shape=[
            jax.ShapeDtypeStruct((N_TOK, K), jnp.float32),
            jax.ShapeDtypeStruct((1, K), jnp.float32),
        ],
        scratch_shapes=[pltpu.VMEM((1, K), jnp.float32)],
        interpret=interpret,
    )(idx)


def _final_body(zq_raw_ref, zn_ref, cnt_ref, zq_ref, loss_ref, ppx_ref):
    zn = zn_ref[...]
    zqn = _rownorm(zq_raw_ref[...])
    zq_ref[...] = zn + (zqn - zn)
    diff = zqn - zn
    m = jnp.sum(diff * diff) / (N_TOK * D)
    loss_ref[0, 0] = BETA * m + m
    p = cnt_ref[...] / N_TOK
    ppx_ref[0, 0] = jnp.exp(-jnp.sum(p * jnp.log(p + 1e-10)))


def _run_final(zq_raw, zn, cnt, interpret=False):
    return pl.pallas_call(
        _final_body,
        in_specs=[
            pl.BlockSpec((N_TOK, D), lambda: (0, 0)),
            pl.BlockSpec((N_TOK, D), lambda: (0, 0)),
            pl.BlockSpec((1, K), lambda: (0, 0)),
        ],
        out_specs=[
            pl.BlockSpec((N_TOK, D), lambda: (0, 0)),
            pl.BlockSpec((1, 1), memory_space=pltpu.SMEM),
            pl.BlockSpec((1, 1), memory_space=pltpu.SMEM),
        ],
        out_shape=[
            jax.ShapeDtypeStruct((N_TOK, D), jnp.float32),
            jax.ShapeDtypeStruct((1, 1), jnp.float32),
            jax.ShapeDtypeStruct((1, 1), jnp.float32),
        ],
        interpret=interpret,
    )(zq_raw, zn, cnt)


def kernel(z, W):
    zt = jnp.transpose(z, (0, 2, 3, 1)).reshape(N_TOK, D)
    zn, idx = _run_argmax(zt, W)
    zq_raw = _sc_gather(W, idx.reshape(1, N_TOK))
    encodings, counts = _run_onehot(idx)
    zq, loss, ppx = _run_final(zq_raw, zn, counts)
    z_q = jnp.transpose(zq.reshape(8, 24, 24, D), (0, 3, 1, 2))
    return (z_q, idx.reshape(N_TOK), loss.reshape(()), encodings, ppx.reshape(()))
